# baseline (speedup 1.0000x reference)
import functools
import math

import jax
import jax.numpy as jnp
from jax import lax
from jax.experimental import pallas as pl
from jax.experimental.pallas import tpu as pltpu
from jax.experimental.pallas import tpu_sc as plsc

_NC = 2
_NS = 16
_NW = _NC * _NS
_L = 16

_LN2 = math.log(2.0)
_X_MAX = 100.0
_ALPHA = 0.75
_SQRT2 = math.sqrt(2.0)


def _ln(x):
    y = lax.bitcast_convert_type(x, jnp.int32)
    e = lax.shift_right_logical(y, 23) - 127
    m = lax.bitcast_convert_type(
        (y & jnp.int32(0x007FFFFF)) | jnp.int32(0x3F800000), jnp.float32)
    big = m > _SQRT2
    m = jnp.where(big, 0.5 * m, m)
    ef = e.astype(jnp.float32) + jnp.where(big, 1.0, 0.0)
    s = (m - 1.0) / (m + 1.0)
    t = s * s
    ln_m = 2.0 * s * (1.0 + t * (1.0 / 3.0 + t * (0.2 + t * (1.0 / 7.0 + t / 9.0))))
    return ef * _LN2 + ln_m


def _sc_glove(c_idx, p_idx, labels, c_embed, c_bias, p_embed, p_bias,
              out, cidx_v, pidx_v, lab_v, ce_v, pe_v, cb_v, pb_v, stage_v,
              tr_v, sem_ce, sem_pe, sem_cb, sem_pb):
    per = lab_v.shape[0]
    nch = cidx_v.shape[0]
    dim = ce_v.shape[1]
    wid = lax.axis_index("s") * _NC + lax.axis_index("c")

    pltpu.sync_copy(c_idx.at[wid], cidx_v)
    pltpu.sync_copy(p_idx.at[wid], pidx_v)
    pltpu.sync_copy(labels.at[wid], lab_v)

    handles = []
    for k in range(nch):
        rows = pl.ds(k * 128, 128)
        handles.append(pltpu.async_copy(c_embed.at[cidx_v.at[k]], ce_v.at[rows], sem_ce))
        handles.append(pltpu.async_copy(p_embed.at[pidx_v.at[k]], pe_v.at[rows], sem_pe))
        handles.append(pltpu.async_copy(c_bias.at[cidx_v.at[k]], cb_v.at[rows], sem_cb))
        handles.append(pltpu.async_copy(p_bias.at[pidx_v.at[k]], pb_v.at[rows], sem_pb))
    for h in handles:
        h.wait()

    lane = lax.iota(jnp.int32, _L)
    lane17 = lane * 17
    nd = dim // _L

    def body(g, acc):
        base = g * _L
        for j in range(_L):
            r = base + j
            prod = ce_v[r, pl.ds(0, _L)] * pe_v[r, pl.ds(0, _L)]
            for k in range(1, nd):
                prod = prod + ce_v[r, pl.ds(k * _L, _L)] * pe_v[r, pl.ds(k * _L, _L)]
            plsc.store_scatter(tr_v, [lane17 + j], prod)
        dots = tr_v[pl.ds(0, _L)]
        for i in range(1, _L):
            dots = dots + tr_v[pl.ds(i * 17, _L)]
        l = lab_v[pl.ds(base, _L)]
        cb = cb_v[pl.ds(base, _L)]
        pb = pb_v[pl.ds(base, _L)]
        lnl = _ln(l)
        w = jnp.minimum(jnp.exp(_ALPHA * (lnl - math.log(_X_MAX))), 1.0)
        diff = dots + cb + pb - lnl
        return acc + w * diff * diff

    acc = lax.fori_loop(0, per // _L, body, jnp.zeros((_L,), jnp.float32))
    stage_v[...] = acc
    pltpu.sync_copy(stage_v, out.at[wid])


def _tc_mean(p_ref, o_ref, *, inv_n):
    o_ref[...] = jnp.sum(p_ref[...], keepdims=True) * inv_n


def kernel(c_data, p_data, labels, c_embed, c_bias, p_embed, p_bias):
    batch = c_data.shape[0]
    vocab, dim = c_embed.shape
    per = batch // _NW
    nch = per // 128

    c3 = c_data.astype(jnp.int32).reshape(_NW, nch, 128)
    p3 = p_data.astype(jnp.int32).reshape(_NW, nch, 128)
    lab2 = labels.reshape(_NW, per)
    cb1 = c_bias.reshape(vocab)
    pb1 = p_bias.reshape(vocab)

    sc = functools.partial(
        pl.kernel,
        mesh=plsc.VectorSubcoreMesh(core_axis_name="c", subcore_axis_name="s"),
        out_type=jax.ShapeDtypeStruct((_NW, _L), jnp.float32),
        compiler_params=pltpu.CompilerParams(
            needs_layout_passes=False, use_tc_tiling_on_sc=False),
        scratch_types=[
            pltpu.VMEM((nch, 128), jnp.int32),
            pltpu.VMEM((nch, 128), jnp.int32),
            pltpu.VMEM((per,), jnp.float32),
            pltpu.VMEM((per, dim), jnp.float32),
            pltpu.VMEM((per, dim), jnp.float32),
            pltpu.VMEM((per,), jnp.float32),
            pltpu.VMEM((per,), jnp.float32),
            pltpu.VMEM((_L,), jnp.float32),
            pltpu.VMEM((_L * 17,), jnp.float32),
            pltpu.SemaphoreType.DMA,
            pltpu.SemaphoreType.DMA,
            pltpu.SemaphoreType.DMA,
            pltpu.SemaphoreType.DMA,
        ],
    )(_sc_glove)
    parts = sc(c3, p3, lab2, c_embed, cb1, p_embed, pb1)

    loss = pl.pallas_call(
        functools.partial(_tc_mean, inv_n=1.0 / batch),
        out_shape=jax.ShapeDtypeStruct((1, 1), jnp.float32),
    )(parts)
    return loss[0, 0]
